# baseline (speedup 1.0000x reference)
import math

import jax
import jax.numpy as jnp
from jax.experimental import pallas as pl

N, C, K, H = 256, 128, 10, 4
DH = C // H
NF = C // 2
NP = 5
EPS = 1e-5

CSIN = (9.99999977e-01, -1.66666476e-01, 8.33289981e-03,
        -1.98008971e-04, 2.59048727e-06)
CCOS = (9.99999953e-01, -4.99999053e-01, 4.16635846e-02,
        -1.38537035e-03, 2.31539157e-05)
CATN = (0.99999992, -0.33332736, 0.19987601, -0.14170616, 0.10526817,
        -0.07282273, 0.04024481, -0.0146438, 0.00250931)


def _dot(a, b, precision=None):
    return jnp.dot(a, b, preferred_element_type=jnp.float32,
                   precision=precision)


def _dgen(a, b, prec=None):
    return jax.lax.dot_general(a, b, (((1,), (1,)), ((), ())),
                               preferred_element_type=jnp.float32,
                               precision=prec)


def _dotT(a, b, prec=None):
    return jax.lax.dot_general(a, b, (((0,), (0,)), ((), ())),
                               preferred_element_type=jnp.float32,
                               precision=prec)


_HI = jax.lax.Precision.HIGHEST


def _gather2(vals, OH):
    hi = vals.astype(jnp.bfloat16).astype(jnp.float32)
    mid = vals - hi
    return _dgen(hi, OH) + _dgen(mid, OH)


def _fused_kernel(coords_ref, feats_ref, W1_ref, W2_ref, W3_ref,
                  p1w_ref, p1b_ref,
                  wq_ref, bq_ref, wk_ref, bk_ref, wv_ref, bv_ref,
                  wp_ref, bp_ref, p2w_ref, p2b_ref, lnw_ref, lnb_ref,
                  out_ref):
    c2 = coords_ref[...]
    f = feats_ref[...]
    ct = jnp.transpose(c2)
    cx = c2[0:1, :]
    cy = c2[1:2, :]
    cxc = ct[:, 0:1]
    cyc = ct[:, 1:2]

    inner = _dot(ct, c2)
    s = jnp.sum(ct * ct, axis=1, keepdims=True)
    d = -2.0 * inner + s + jnp.transpose(s)
    d = jnp.maximum(d, 1e-12)

    iota = jax.lax.broadcasted_iota(jnp.int32, (N, N), 1)
    work = d
    ohs = []
    for t in range(K + 1):
        mn = jnp.min(work, axis=1, keepdims=True)
        cand = jnp.where(work == mn, iota, jnp.int32(N))
        sel = jnp.min(cand, axis=1, keepdims=True)
        oh = (iota == sel).astype(jnp.float32)
        work = jnp.where(oh > 0.0, jnp.float32(1e30), work)
        if t >= 1:
            ohs.append(oh)
    OHcat = jnp.concatenate(ohs, axis=0)

    fplus = jnp.concatenate([f, c2], axis=0)
    NB = _gather2(fplus, OHcat)
    W1 = W1_ref[...]
    ftile = jnp.concatenate([f] * K, axis=1)
    base1 = _dot(W1[:, :C], f)
    h1 = _dot(W1[:, C:], NB[:C, :] - ftile) + jnp.concatenate([base1] * K, axis=1)
    m1 = jnp.mean(h1, axis=1, keepdims=True)
    dev1 = h1 - m1
    v1 = jnp.mean(dev1 * dev1, axis=1, keepdims=True)
    h1 = dev1 / jnp.sqrt(v1 + EPS)
    h1 = jnp.where(h1 >= 0.0, h1, 0.2 * h1)
    feats1 = h1[:, :N]
    for j in range(1, K):
        feats1 = jnp.maximum(feats1, h1[:, j * N:(j + 1) * N])

    NB2 = _gather2(feats1, OHcat)
    W2 = W2_ref[...]
    f1tile = jnp.concatenate([feats1] * K, axis=1)
    base2 = _dot(W2[:, :C], feats1)
    h2 = _dot(W2[:, C:], NB2 - f1tile) + jnp.concatenate([base2] * K, axis=1)
    m2 = jnp.mean(h2, axis=1, keepdims=True)
    dev2 = h2 - m2
    v2 = jnp.mean(dev2 * dev2, axis=1, keepdims=True)
    h2 = dev2 / jnp.sqrt(v2 + EPS)
    h2 = jnp.where(h2 >= 0.0, h2, 0.2 * h2)
    x2m = h2[:, :N]
    for j in range(1, K):
        x2m = jnp.maximum(x2m, h2[:, j * N:(j + 1) * N])

    x3in = jnp.concatenate([f, feats1, x2m], axis=0)
    h3 = _dot(W3_ref[...], x3in)
    m3 = jnp.mean(h3, axis=1, keepdims=True)
    dev3 = h3 - m3
    v3 = jnp.mean(dev3 * dev3, axis=1, keepdims=True)
    h3 = dev3 / jnp.sqrt(v3 + EPS)
    x3cn = jnp.where(h3 >= 0.0, h3, 0.2 * h3)
    x3 = jnp.transpose(x3cn)

    qT = _dot(wq_ref[...], x3cn) + bq_ref[...]
    kkT = _dot(wk_ref[...], x3cn) + bk_ref[...]
    v = _dgen(x3, wv_ref[...]) + bv_ref[...]

    axT = cxc - cx
    ayT = cyc - cy
    maT = jnp.sqrt(axT * axT + ayT * ayT)
    A0, A1, A2, A3, A4, A5, A6, A7, A8 = CATN
    thetas = []
    for j in range(K):
        rxr = NB[C:C + 1, j * N:(j + 1) * N] - cx
        ryr = NB[C + 1:C + 2, j * N:(j + 1) * N] - cy
        dotj = axT * rxr + ayT * ryr
        den = jnp.sqrt(rxr * rxr + ryr * ryr) * maT
        r = jnp.sqrt(den * den + dotj * dotj)
        tt = dotj / (den + r)
        tt = jnp.where(r > 0.0, tt, 0.0)
        t2 = tt * tt
        poly = A0 + t2 * (A1 + t2 * (A2 + t2 * (A3 + t2 * (
            A4 + t2 * (A5 + t2 * (A6 + t2 * (A7 + t2 * A8)))))))
        thetas.append(tt * poly)

    Ts = []
    curs = thetas
    Ts.append(sum(curs[1:], curs[0]))
    for p in range(2, 2 * NP):
        curs = [c * t for c, t in zip(curs, thetas)]
        Ts.append(sum(curs[1:], curs[0]))

    wp = wp_ref[...]
    Afull = _dot(wp, p1w_ref[...])
    bcc = _dot(wp, p1b_ref[...]) + bp_ref[...]
    jd2 = jax.lax.broadcasted_iota(jnp.int32, (C, 1), 0)
    is_sin = (jd2 % 2) == 0
    dcol = jnp.exp((jd2 // 2).astype(jnp.float32)
                   * (-2.0 * math.log(10000.0) / C))
    d2 = dcol * dcol
    sincols = []
    cur = jnp.where(is_sin, dcol, 0.0)
    for i in range(NP):
        sincols.append(cur * CSIN[i])
        cur = cur * d2
    Dsin = jnp.concatenate(sincols, axis=1)
    coscols = []
    cur = jnp.where(is_sin, 0.0, 1.0)
    for i in range(NP):
        coscols.append(cur * CCOS[i])
        cur = cur * d2
    Dcos = jnp.concatenate(coscols, axis=1)

    gs_rows = []
    gc_rows = []
    for h in range(H):
        sl = slice(h * DH, (h + 1) * DH)
        Bsin = _dot(Afull[sl, :], Dsin, precision=_HI)
        Bcos = _dot(Afull[sl, :], Dcos, precision=_HI)
        Bcos = jnp.concatenate([Bcos[:, 0:1] + bcc[sl, :], Bcos[:, 1:]], axis=1)
        gs_rows.append(_dotT(Bsin, qT[sl, :], _HI))
        gc_rows.append(_dotT(Bcos, qT[sl, :], _HI))
    gsin = jnp.concatenate(gs_rows, axis=0)
    gcos = jnp.concatenate(gc_rows, axis=0)

    scale = 1.0 / math.sqrt(DH)
    pscale = scale / K
    heads = []
    for h in range(H):
        sl = slice(h * DH, (h + 1) * DH)
        acc = gcos[NP * h:NP * h + 1, :] * jnp.float32(K)
        for i in range(NP):
            acc = acc + gsin[NP * h + i:NP * h + i + 1, :] * Ts[2 * i]
            if i > 0:
                acc = acc + gcos[NP * h + i:NP * h + i + 1, :] * Ts[2 * i - 1]
        seT = _dotT(kkT[sl, :], qT[sl, :])
        z = seT * scale + acc * pscale
        z = z - jnp.max(z, axis=0, keepdims=True)
        e = jnp.exp(z)
        pT = e / jnp.sum(e, axis=0, keepdims=True)
        heads.append(_dotT(pT, v[:, sl]))

    hidden = jnp.concatenate(heads, axis=1)
    x4 = _dgen(hidden, p2w_ref[...]) + p2b_ref[...]
    y = x3 + x4
    m = jnp.mean(y, axis=1, keepdims=True)
    var = jnp.mean((y - m) ** 2, axis=1, keepdims=True)
    y = (y - m) / jnp.sqrt(var + EPS) * lnw_ref[...] + lnb_ref[...]
    out_ref[...] = jnp.transpose(y)


def kernel(coords, features, W1, W2, W3, p1_w, p1_b, wq, bq, wk, bk, wv, bv,
           wp, bp, p2_w, p2_b, ln_w, ln_b):
    c2 = coords[0]
    f = features[0]
    row = lambda b: b.reshape(1, C)
    col = lambda b: b.reshape(C, 1)

    out = pl.pallas_call(
        _fused_kernel,
        out_shape=jax.ShapeDtypeStruct((C, N), jnp.float32),
    )(c2, f, W1, W2, W3, p1_w, col(p1_b),
      wq, col(bq), wk, col(bk), wv, row(bv), wp, col(bp),
      p2_w, row(p2_b), row(ln_w), row(ln_b))
    return out[None, :, :]


# speedup vs baseline: 1.0387x; 1.0387x over previous
import math

import jax
import jax.numpy as jnp
from jax.experimental import pallas as pl

N, C, K, H = 256, 128, 10, 4
DH = C // H
NF = C // 2
NP = 4
EPS = 1e-5

CSIN = (9.99996616e-01, -1.66648283e-01, 8.30632441e-03,
        -1.83636349e-04)
CCOS = (0.99999329, -0.49991243, 0.04148774,
        -0.00127121)
CATN = (0.9999961, -0.33317342, 0.19807598, -0.13232579, 0.07961075,
        -0.03359375, 0.00680854)


def _dot(a, b, precision=None):
    return jnp.dot(a, b, preferred_element_type=jnp.float32,
                   precision=precision)


def _dgen(a, b, prec=None):
    return jax.lax.dot_general(a, b, (((1,), (1,)), ((), ())),
                               preferred_element_type=jnp.float32,
                               precision=prec)


def _dotT(a, b, prec=None):
    return jax.lax.dot_general(a, b, (((0,), (0,)), ((), ())),
                               preferred_element_type=jnp.float32,
                               precision=prec)


_HI = jax.lax.Precision.HIGHEST


def _gather2(vals, OH):
    hi = vals.astype(jnp.bfloat16).astype(jnp.float32)
    mid = vals - hi
    return _dgen(hi, OH) + _dgen(mid, OH)


def _fused_kernel(coords_ref, feats_ref, W1_ref, W2_ref, W3_ref,
                  p1w_ref, p1b_ref,
                  wq_ref, bq_ref, wk_ref, bk_ref, wv_ref, bv_ref,
                  wp_ref, bp_ref, p2w_ref, p2b_ref, lnw_ref, lnb_ref,
                  out_ref):
    c2 = coords_ref[...]
    f = feats_ref[...]
    ct = jnp.transpose(c2)
    cx = c2[0:1, :]
    cy = c2[1:2, :]
    cxc = ct[:, 0:1]
    cyc = ct[:, 1:2]

    inner = _dot(ct, c2)
    s = jnp.sum(ct * ct, axis=1, keepdims=True)
    d = -2.0 * inner + s + jnp.transpose(s)
    d = jnp.maximum(d, 1e-12)

    iota = jax.lax.broadcasted_iota(jnp.int32, (N, N), 1)
    work = d
    ohs = []
    for t in range(K + 1):
        mn = jnp.min(work, axis=1, keepdims=True)
        cand = jnp.where(work == mn, iota, jnp.int32(N))
        sel = jnp.min(cand, axis=1, keepdims=True)
        oh = (iota == sel).astype(jnp.float32)
        work = jnp.where(oh > 0.0, jnp.float32(1e30), work)
        if t >= 1:
            ohs.append(oh)
    OHcat = jnp.concatenate(ohs, axis=0)

    fplus = jnp.concatenate([f, c2], axis=0)
    NB = _gather2(fplus, OHcat)
    W1 = W1_ref[...]
    ftile = jnp.concatenate([f] * K, axis=1)
    base1 = _dot(W1[:, :C], f)
    h1 = _dot(W1[:, C:], NB[:C, :] - ftile) + jnp.concatenate([base1] * K, axis=1)
    m1 = jnp.mean(h1, axis=1, keepdims=True)
    dev1 = h1 - m1
    v1 = jnp.mean(dev1 * dev1, axis=1, keepdims=True)
    h1 = dev1 / jnp.sqrt(v1 + EPS)
    h1 = jnp.where(h1 >= 0.0, h1, 0.2 * h1)
    feats1 = h1[:, :N]
    for j in range(1, K):
        feats1 = jnp.maximum(feats1, h1[:, j * N:(j + 1) * N])

    NB2 = _gather2(feats1, OHcat)
    W2 = W2_ref[...]
    f1tile = jnp.concatenate([feats1] * K, axis=1)
    base2 = _dot(W2[:, :C], feats1)
    h2 = _dot(W2[:, C:], NB2 - f1tile) + jnp.concatenate([base2] * K, axis=1)
    m2 = jnp.mean(h2, axis=1, keepdims=True)
    dev2 = h2 - m2
    v2 = jnp.mean(dev2 * dev2, axis=1, keepdims=True)
    h2 = dev2 / jnp.sqrt(v2 + EPS)
    h2 = jnp.where(h2 >= 0.0, h2, 0.2 * h2)
    x2m = h2[:, :N]
    for j in range(1, K):
        x2m = jnp.maximum(x2m, h2[:, j * N:(j + 1) * N])

    x3in = jnp.concatenate([f, feats1, x2m], axis=0)
    h3 = _dot(W3_ref[...], x3in)
    m3 = jnp.mean(h3, axis=1, keepdims=True)
    dev3 = h3 - m3
    v3 = jnp.mean(dev3 * dev3, axis=1, keepdims=True)
    h3 = dev3 / jnp.sqrt(v3 + EPS)
    x3cn = jnp.where(h3 >= 0.0, h3, 0.2 * h3)
    x3 = jnp.transpose(x3cn)

    qT = _dot(wq_ref[...], x3cn) + bq_ref[...]
    kkT = _dot(wk_ref[...], x3cn) + bk_ref[...]
    v = _dgen(x3, wv_ref[...]) + bv_ref[...]

    axT = cxc - cx
    ayT = cyc - cy
    maT = jnp.sqrt(axT * axT + ayT * ayT)
    A0, A1, A2, A3, A4, A5, A6 = CATN
    thetas = []
    for j in range(K):
        rxr = NB[C:C + 1, j * N:(j + 1) * N] - cx
        ryr = NB[C + 1:C + 2, j * N:(j + 1) * N] - cy
        dotj = axT * rxr + ayT * ryr
        den = jnp.sqrt(rxr * rxr + ryr * ryr) * maT
        r = jnp.sqrt(den * den + dotj * dotj)
        tt = dotj / (den + r)
        tt = jnp.where(r > 0.0, tt, 0.0)
        t2 = tt * tt
        poly = A0 + t2 * (A1 + t2 * (A2 + t2 * (A3 + t2 * (
            A4 + t2 * (A5 + t2 * A6)))))
        thetas.append(tt * poly)

    Ts = []
    curs = thetas
    Ts.append(sum(curs[1:], curs[0]))
    for p in range(2, 2 * NP):
        curs = [c * t for c, t in zip(curs, thetas)]
        Ts.append(sum(curs[1:], curs[0]))

    wp = wp_ref[...]
    Afull = _dot(wp, p1w_ref[...])
    bcc = _dot(wp, p1b_ref[...]) + bp_ref[...]
    jd2 = jax.lax.broadcasted_iota(jnp.int32, (C, 1), 0)
    is_sin = (jd2 % 2) == 0
    dcol = jnp.exp((jd2 // 2).astype(jnp.float32)
                   * (-2.0 * math.log(10000.0) / C))
    d2 = dcol * dcol
    sincols = []
    cur = jnp.where(is_sin, dcol, 0.0)
    for i in range(NP):
        sincols.append(cur * CSIN[i])
        cur = cur * d2
    Dsin = jnp.concatenate(sincols, axis=1)
    coscols = []
    cur = jnp.where(is_sin, 0.0, 1.0)
    for i in range(NP):
        coscols.append(cur * CCOS[i])
        cur = cur * d2
    Dcos = jnp.concatenate(coscols, axis=1)

    gs_rows = []
    gc_rows = []
    for h in range(H):
        sl = slice(h * DH, (h + 1) * DH)
        Bsin = _dot(Afull[sl, :], Dsin, precision=_HI)
        Bcos = _dot(Afull[sl, :], Dcos, precision=_HI)
        Bcos = jnp.concatenate([Bcos[:, 0:1] + bcc[sl, :], Bcos[:, 1:]], axis=1)
        gs_rows.append(_dotT(Bsin, qT[sl, :], _HI))
        gc_rows.append(_dotT(Bcos, qT[sl, :], _HI))
    gsin = jnp.concatenate(gs_rows, axis=0)
    gcos = jnp.concatenate(gc_rows, axis=0)

    scale = 1.0 / math.sqrt(DH)
    pscale = scale / K
    heads = []
    for h in range(H):
        sl = slice(h * DH, (h + 1) * DH)
        acc = gcos[NP * h:NP * h + 1, :] * jnp.float32(K)
        for i in range(NP):
            acc = acc + gsin[NP * h + i:NP * h + i + 1, :] * Ts[2 * i]
            if i > 0:
                acc = acc + gcos[NP * h + i:NP * h + i + 1, :] * Ts[2 * i - 1]
        seT = _dotT(kkT[sl, :], qT[sl, :])
        z = seT * scale + acc * pscale
        z = z - jnp.max(z, axis=0, keepdims=True)
        e = jnp.exp(z)
        pT = e / jnp.sum(e, axis=0, keepdims=True)
        heads.append(_dotT(pT, v[:, sl]))

    hidden = jnp.concatenate(heads, axis=1)
    x4 = _dgen(hidden, p2w_ref[...]) + p2b_ref[...]
    y = x3 + x4
    m = jnp.mean(y, axis=1, keepdims=True)
    var = jnp.mean((y - m) ** 2, axis=1, keepdims=True)
    y = (y - m) / jnp.sqrt(var + EPS) * lnw_ref[...] + lnb_ref[...]
    out_ref[...] = jnp.transpose(y)


def kernel(coords, features, W1, W2, W3, p1_w, p1_b, wq, bq, wk, bk, wv, bv,
           wp, bp, p2_w, p2_b, ln_w, ln_b):
    c2 = coords[0]
    f = features[0]
    row = lambda b: b.reshape(1, C)
    col = lambda b: b.reshape(C, 1)

    out = pl.pallas_call(
        _fused_kernel,
        out_shape=jax.ShapeDtypeStruct((C, N), jnp.float32),
    )(c2, f, W1, W2, W3, p1_w, col(p1_b),
      wq, col(bq), wk, col(bk), wv, row(bv), wp, col(bp),
      p2_w, row(p2_b), row(ln_w), row(ln_b))
    return out[None, :, :]
